# Initial kernel scaffold; baseline (speedup 1.0000x reference)
import functools

import jax
import jax.numpy as jnp
from jax import lax
from jax.experimental import pallas as pl
from jax.experimental.pallas import tpu as pltpu, tpu_sc as plsc

EPS = 1e-5
CHUNK = 128


def _message_pass_kernel(n_nodes, d, edges_per_tile, num_subcores):
    n_chunks = edges_per_tile // CHUNK
    rows_per_tile = n_nodes // num_subcores
    wb = rows_per_tile // 5

    def body(var_hbm, constr_hbm, src_hbm, dst_hbm, attr_hbm,
             cagg_hbm, vagg_hbm,
             acc, idx_g, idx_s, attr_v, rows, sem):
        c = lax.axis_index("c")
        s = lax.axis_index("s")

        def zero_row(r, carry):
            for g in range(d // 16):
                rows[r, pl.ds(g * 16, 16)] = jnp.zeros((16,), jnp.float32)
            return carry
        lax.fori_loop(0, CHUNK, zero_row, 0)
        for j in range(rows_per_tile // wb):
            pltpu.sync_copy(rows.at[pl.ds(0, wb)],
                            acc.at[pl.ds(s * rows_per_tile + j * wb, wb)])
        plsc.subcore_barrier()

        def chunk_body(k, carry):
            base = s * edges_per_tile + k * CHUNK

            @pl.when(c == 0)
            def _():
                pltpu.sync_copy(src_hbm.at[pl.ds(base, CHUNK)], idx_g)
                pltpu.sync_copy(dst_hbm.at[pl.ds(base, CHUNK)], idx_s)

            @pl.when(c == 1)
            def _():
                pltpu.sync_copy(dst_hbm.at[pl.ds(base, CHUNK)], idx_g)
                pltpu.sync_copy(src_hbm.at[pl.ds(base, CHUNK)], idx_s)

            pltpu.sync_copy(attr_hbm.at[pl.ds(base, CHUNK)], attr_v)

            @pl.when(c == 0)
            def _():
                pltpu.async_copy(var_hbm.at[idx_g], rows, sem).wait()

            @pl.when(c == 1)
            def _():
                pltpu.async_copy(constr_hbm.at[idx_g], rows, sem).wait()

            def scale_row(e, carry2):
                a = attr_v[e]
                for g in range(d // 16):
                    rows[e, pl.ds(g * 16, 16)] = rows[e, pl.ds(g * 16, 16)] * a
                return carry2
            lax.fori_loop(0, CHUNK, scale_row, 0)

            pltpu.sync_copy(rows, acc.at[idx_s], add=True)
            return carry
        lax.fori_loop(0, n_chunks, chunk_body, 0)
        plsc.subcore_barrier()

        for j in range(rows_per_tile // wb):
            r0 = s * rows_per_tile + j * wb
            pltpu.sync_copy(acc.at[pl.ds(r0, wb)], rows.at[pl.ds(0, wb)])

            @pl.when(c == 0)
            def _():
                pltpu.sync_copy(rows.at[pl.ds(0, wb)], cagg_hbm.at[pl.ds(r0, wb)])

            @pl.when(c == 1)
            def _():
                pltpu.sync_copy(rows.at[pl.ds(0, wb)], vagg_hbm.at[pl.ds(r0, wb)])

    return body


def _message_pass(var_feats, constr_feats, src, dst, attr):
    n_nodes, d = var_feats.shape
    info = plsc.get_sparse_core_info()
    ns = info.num_subcores
    e_total = src.shape[0]
    edges_per_tile = e_total // ns

    mesh = plsc.VectorSubcoreMesh(core_axis_name="c", subcore_axis_name="s")
    body = _message_pass_kernel(n_nodes, d, edges_per_tile, ns)
    out_t = jax.ShapeDtypeStruct((n_nodes, d), jnp.float32)
    k = pl.kernel(
        body,
        out_type=(out_t, out_t),
        mesh=mesh,
        scratch_types=[
            pltpu.VMEM_SHARED((n_nodes, d), jnp.float32),
            pltpu.VMEM((CHUNK,), jnp.int32),
            pltpu.VMEM((CHUNK,), jnp.int32),
            pltpu.VMEM((CHUNK,), jnp.float32),
            pltpu.VMEM((CHUNK, d), jnp.float32),
            pltpu.SemaphoreType.DMA,
        ],
    )
    return k(var_feats, constr_feats, src, dst, attr)


def _mlp_body(x_ref, agg_ref, w1a_ref, w1b_ref, b1_ref, g_ref, bt_ref,
              w2_ref, b2_ref, out_ref):
    x = x_ref[...]
    h = jnp.dot(x, w1a_ref[...], preferred_element_type=jnp.float32)
    h = h + jnp.dot(agg_ref[...], w1b_ref[...], preferred_element_type=jnp.float32)
    h = h + b1_ref[...]
    mu = jnp.mean(h, axis=0, keepdims=True)
    var = jnp.mean((h - mu) ** 2, axis=0, keepdims=True)
    hn = (h - mu) * (g_ref[...] * lax.rsqrt(var + EPS)) + bt_ref[...]
    hr = jnp.maximum(hn, 0.0)
    out_ref[...] = x + jnp.dot(hr, w2_ref[...], preferred_element_type=jnp.float32) + b2_ref[...]


def _mlp_update(x, agg, W1, b1, g, bt, W2, b2):
    n, d = x.shape
    w1a = W1[:, :d].T
    w1b = W1[:, d:].T
    return pl.pallas_call(
        _mlp_body,
        out_shape=jax.ShapeDtypeStruct((n, d), jnp.float32),
    )(x, agg, w1a, w1b, b1.reshape(1, -1), g.reshape(1, -1),
      bt.reshape(1, -1), W2.T, b2.reshape(1, -1))


def kernel(var_feats, constr_feats, edge_index, edge_attr,
           W1, b1, g1, bt1, W2, b2, W3, b3, g2, bt2, W4, b4):
    n_edges = edge_index.shape[1]
    info = plsc.get_sparse_core_info()
    ns = info.num_subcores
    per_tile = -(-n_edges // (ns * CHUNK)) * CHUNK
    e_pad = per_tile * ns
    pad = e_pad - n_edges
    src = jnp.pad(edge_index[0], (0, pad))
    dst = jnp.pad(edge_index[1], (0, pad))
    attr = jnp.pad(edge_attr, (0, pad))

    constr_agg, var_agg = _message_pass(var_feats, constr_feats, src, dst, attr)
    var_updated = _mlp_update(var_feats, var_agg, W1, b1, g1, bt1, W2, b2)
    constr_updated = _mlp_update(constr_feats, constr_agg, W3, b3, g2, bt2, W4, b4)
    return (var_updated, constr_updated)


# speedup vs baseline: 4.5794x; 4.5794x over previous
import functools

import jax
import jax.numpy as jnp
from jax import lax
from jax.experimental import pallas as pl
from jax.experimental.pallas import tpu as pltpu, tpu_sc as plsc

EPS = 1e-5
CHUNK = 128


def _message_pass_kernel(n_nodes, d, edges_per_tile, num_subcores):
    n_chunks = edges_per_tile // CHUNK
    W = 80
    n_wchunks = n_nodes // W
    max_wchunks_per_tile = -(-n_wchunks // num_subcores)

    def body(table_hbm, src_hbm, dst_hbm, dsto_hbm, attr_hbm,
             cagg_hbm, vagg_hbm,
             acc, idx_g, idx_s, attr_v, rows, sem):
        c = lax.axis_index("c")
        s = lax.axis_index("s")

        def zero_row(r, carry):
            for g in range(d // 16):
                rows[r, pl.ds(g * 16, 16)] = jnp.zeros((16,), jnp.float32)
            return carry
        lax.fori_loop(0, CHUNK, zero_row, 0)

        def zero_chunk(i, carry):
            cid = s + i * num_subcores

            @pl.when(cid < n_wchunks)
            def _():
                r0 = pl.multiple_of(cid * W, 8)
                pltpu.sync_copy(rows.at[pl.ds(0, W)], acc.at[pl.ds(r0, W)])
            return carry
        lax.fori_loop(0, max_wchunks_per_tile, zero_chunk, 0)
        plsc.subcore_barrier()

        def chunk_body(k, carry):
            base = pl.multiple_of(s * edges_per_tile + k * CHUNK, CHUNK)

            @pl.when(c == 0)
            def _():
                pltpu.sync_copy(src_hbm.at[pl.ds(base, CHUNK)], idx_g)
                pltpu.sync_copy(dst_hbm.at[pl.ds(base, CHUNK)], idx_s)

            @pl.when(c == 1)
            def _():
                pltpu.sync_copy(dsto_hbm.at[pl.ds(base, CHUNK)], idx_g)
                pltpu.sync_copy(src_hbm.at[pl.ds(base, CHUNK)], idx_s)

            pltpu.sync_copy(attr_hbm.at[pl.ds(base, CHUNK)], attr_v)

            pltpu.async_copy(table_hbm.at[idx_g], rows, sem).wait()

            def scale_group(gi, carry2):
                a16 = attr_v[pl.ds(gi * 16, 16)]
                for j in range(16):
                    a = a16[j]
                    e = gi * 16 + j
                    for g in range(d // 16):
                        rows[e, pl.ds(g * 16, 16)] = rows[e, pl.ds(g * 16, 16)] * a
                return carry2
            lax.fori_loop(0, CHUNK // 16, scale_group, 0)

            pltpu.sync_copy(rows, acc.at[idx_s], add=True)
            return carry
        lax.fori_loop(0, n_chunks, chunk_body, 0)
        plsc.subcore_barrier()

        def wb_chunk(i, carry):
            cid = s + i * num_subcores

            @pl.when(cid < n_wchunks)
            def _():
                r0 = pl.multiple_of(cid * W, 8)
                pltpu.sync_copy(acc.at[pl.ds(r0, W)], rows.at[pl.ds(0, W)])

                @pl.when(c == 0)
                def _():
                    pltpu.sync_copy(rows.at[pl.ds(0, W)], cagg_hbm.at[pl.ds(r0, W)])

                @pl.when(c == 1)
                def _():
                    pltpu.sync_copy(rows.at[pl.ds(0, W)], vagg_hbm.at[pl.ds(r0, W)])
            return carry
        lax.fori_loop(0, max_wchunks_per_tile, wb_chunk, 0)

    return body


def _message_pass(var_feats, constr_feats, src, dst, attr):
    n_nodes, d = var_feats.shape
    info = plsc.get_sparse_core_info()
    ns = info.num_subcores
    e_total = src.shape[0]
    edges_per_tile = e_total // ns
    table = jnp.concatenate([var_feats, constr_feats], axis=0)
    dst_off = dst + n_nodes

    mesh = plsc.VectorSubcoreMesh(core_axis_name="c", subcore_axis_name="s")
    body = _message_pass_kernel(n_nodes, d, edges_per_tile, ns)
    out_t = jax.ShapeDtypeStruct((n_nodes, d), jnp.float32)
    k = pl.kernel(
        body,
        out_type=(out_t, out_t),
        mesh=mesh,
        scratch_types=[
            pltpu.VMEM_SHARED((n_nodes, d), jnp.float32),
            pltpu.VMEM((CHUNK,), jnp.int32),
            pltpu.VMEM((CHUNK,), jnp.int32),
            pltpu.VMEM((CHUNK,), jnp.float32),
            pltpu.VMEM((CHUNK, d), jnp.float32),
            pltpu.SemaphoreType.DMA,
        ],
    )
    return k(table, src, dst, dst_off, attr)


def _mlp_body(x_ref, agg_ref, w1a_ref, w1b_ref, b1_ref, g_ref, bt_ref,
              w2_ref, b2_ref, out_ref):
    x = x_ref[...]
    h = jnp.dot(x, w1a_ref[...], preferred_element_type=jnp.float32)
    h = h + jnp.dot(agg_ref[...], w1b_ref[...], preferred_element_type=jnp.float32)
    h = h + b1_ref[...]
    mu = jnp.mean(h, axis=0, keepdims=True)
    var = jnp.mean((h - mu) ** 2, axis=0, keepdims=True)
    hn = (h - mu) * (g_ref[...] * lax.rsqrt(var + EPS)) + bt_ref[...]
    hr = jnp.maximum(hn, 0.0)
    out_ref[...] = x + jnp.dot(hr, w2_ref[...], preferred_element_type=jnp.float32) + b2_ref[...]


def _mlp_update(x, agg, W1, b1, g, bt, W2, b2):
    n, d = x.shape
    w1a = W1[:, :d].T
    w1b = W1[:, d:].T
    return pl.pallas_call(
        _mlp_body,
        out_shape=jax.ShapeDtypeStruct((n, d), jnp.float32),
    )(x, agg, w1a, w1b, b1.reshape(1, -1), g.reshape(1, -1),
      bt.reshape(1, -1), W2.T, b2.reshape(1, -1))


def kernel(var_feats, constr_feats, edge_index, edge_attr,
           W1, b1, g1, bt1, W2, b2, W3, b3, g2, bt2, W4, b4):
    n_edges = edge_index.shape[1]
    info = plsc.get_sparse_core_info()
    ns = info.num_subcores
    per_tile = -(-n_edges // (ns * CHUNK)) * CHUNK
    e_pad = per_tile * ns
    pad = e_pad - n_edges
    src = jnp.pad(edge_index[0], (0, pad))
    dst = jnp.pad(edge_index[1], (0, pad))
    attr = jnp.pad(edge_attr, (0, pad))

    constr_agg, var_agg = _message_pass(var_feats, constr_feats, src, dst, attr)
    var_updated = _mlp_update(var_feats, var_agg, W1, b1, g1, bt1, W2, b2)
    constr_updated = _mlp_update(constr_feats, constr_agg, W3, b3, g2, bt2, W4, b4)
    return (var_updated, constr_updated)


# speedup vs baseline: 5.9267x; 1.2942x over previous
import jax
import jax.numpy as jnp
from jax import lax
from jax.experimental import pallas as pl
from jax.experimental.pallas import tpu as pltpu, tpu_sc as plsc

EPS = 1e-5
CHUNK = 128


def _message_pass_kernel(n_nodes, d, chunks_per_tile, num_subcores):
    W = 80
    n_wchunks = n_nodes // W
    max_wchunks_per_tile = -(-n_wchunks // num_subcores)
    ng = d // 16

    def body(table_hbm, p0_hbm, p1_hbm, attr_hbm,
             cagg_hbm, vagg_hbm,
             acc, buf0, buf1, av0, av1, rows0, rows1, sem0, sem1):
        c = lax.axis_index("c")
        s = lax.axis_index("s")

        def load_buf(buf, av, g):
            @pl.when(c == 0)
            def _():
                pltpu.sync_copy(p0_hbm.at[g], buf)

            @pl.when(c == 1)
            def _():
                pltpu.sync_copy(p1_hbm.at[g], buf)
            pltpu.sync_copy(attr_hbm.at[g], av)

        def start_gather(buf, rows, sem):
            pltpu.async_copy(table_hbm.at[buf.at[0]], rows, sem)

        def finish_chunk(buf, av, rows, sem):
            pltpu.make_async_copy(table_hbm.at[buf.at[0]], rows, sem).wait()

            def scale_group(gi, carry2):
                a16 = av[pl.ds(gi * 16, 16)]
                for j in range(16):
                    a = a16[j]
                    e = gi * 16 + j
                    for g in range(ng):
                        rows[e, pl.ds(g * 16, 16)] = rows[e, pl.ds(g * 16, 16)] * a
                return carry2
            lax.fori_loop(0, CHUNK // 16, scale_group, 0)
            pltpu.sync_copy(rows, acc.at[buf.at[1]], add=True)

        def zero_row(r, carry):
            for g in range(ng):
                rows0[r, pl.ds(g * 16, 16)] = jnp.zeros((16,), jnp.float32)
            return carry
        lax.fori_loop(0, CHUNK, zero_row, 0)

        def zero_chunk(i, carry):
            cid = s + i * num_subcores

            @pl.when(cid < n_wchunks)
            def _():
                r0 = pl.multiple_of(cid * W, 8)
                pltpu.sync_copy(rows0.at[pl.ds(0, W)], acc.at[pl.ds(r0, W)])
            return carry
        lax.fori_loop(0, max_wchunks_per_tile, zero_chunk, 0)
        plsc.subcore_barrier()

        g_base = s * chunks_per_tile
        load_buf(buf0, av0, g_base)
        start_gather(buf0, rows0, sem0)

        def pair(i, carry):
            g = g_base + 2 * i
            load_buf(buf1, av1, g + 1)
            start_gather(buf1, rows1, sem1)
            finish_chunk(buf0, av0, rows0, sem0)

            @pl.when(2 * i + 2 < chunks_per_tile)
            def _():
                load_buf(buf0, av0, g + 2)
                start_gather(buf0, rows0, sem0)
            finish_chunk(buf1, av1, rows1, sem1)
            return carry
        lax.fori_loop(0, chunks_per_tile // 2, pair, 0)
        plsc.subcore_barrier()

        def wb_chunk(i, carry):
            cid = s + i * num_subcores

            @pl.when(cid < n_wchunks)
            def _():
                r0 = pl.multiple_of(cid * W, 8)
                pltpu.sync_copy(acc.at[pl.ds(r0, W)], rows0.at[pl.ds(0, W)])

                @pl.when(c == 0)
                def _():
                    pltpu.sync_copy(rows0.at[pl.ds(0, W)], cagg_hbm.at[pl.ds(r0, W)])

                @pl.when(c == 1)
                def _():
                    pltpu.sync_copy(rows0.at[pl.ds(0, W)], vagg_hbm.at[pl.ds(r0, W)])
            return carry
        lax.fori_loop(0, max_wchunks_per_tile, wb_chunk, 0)

    return body


def _message_pass(var_feats, constr_feats, src, dst, attr):
    n_nodes, d = var_feats.shape
    info = plsc.get_sparse_core_info()
    ns = info.num_subcores
    e_total = src.shape[0]
    n_chunks_tot = e_total // CHUNK
    chunks_per_tile = n_chunks_tot // ns

    table = jnp.concatenate([var_feats, constr_feats], axis=0)
    ar = attr.reshape(n_chunks_tot, CHUNK)
    sr = src.reshape(n_chunks_tot, CHUNK)
    dr = dst.reshape(n_chunks_tot, CHUNK)
    packed0 = jnp.stack([sr, dr], axis=1)
    packed1 = jnp.stack([dr + n_nodes, sr], axis=1)

    mesh = plsc.VectorSubcoreMesh(core_axis_name="c", subcore_axis_name="s")
    body = _message_pass_kernel(n_nodes, d, chunks_per_tile, ns)
    out_t = jax.ShapeDtypeStruct((n_nodes, d), jnp.float32)
    k = pl.kernel(
        body,
        out_type=(out_t, out_t),
        mesh=mesh,
        scratch_types=[
            pltpu.VMEM_SHARED((n_nodes, d), jnp.float32),
            pltpu.VMEM((2, CHUNK), jnp.int32),
            pltpu.VMEM((2, CHUNK), jnp.int32),
            pltpu.VMEM((CHUNK,), jnp.float32),
            pltpu.VMEM((CHUNK,), jnp.float32),
            pltpu.VMEM((CHUNK, d), jnp.float32),
            pltpu.VMEM((CHUNK, d), jnp.float32),
            pltpu.SemaphoreType.DMA,
            pltpu.SemaphoreType.DMA,
        ],
    )
    return k(table, packed0, packed1, ar)


def _mlp_body(x_ref, agg_ref, w1a_ref, w1b_ref, b1_ref, g_ref, bt_ref,
              w2_ref, b2_ref, out_ref):
    x = x_ref[...]
    h = jnp.dot(x, w1a_ref[...], preferred_element_type=jnp.float32)
    h = h + jnp.dot(agg_ref[...], w1b_ref[...], preferred_element_type=jnp.float32)
    h = h + b1_ref[...]
    mu = jnp.mean(h, axis=0, keepdims=True)
    var = jnp.mean((h - mu) ** 2, axis=0, keepdims=True)
    hn = (h - mu) * (g_ref[...] * lax.rsqrt(var + EPS)) + bt_ref[...]
    hr = jnp.maximum(hn, 0.0)
    out_ref[...] = x + jnp.dot(hr, w2_ref[...], preferred_element_type=jnp.float32) + b2_ref[...]


def _mlp_update(x, agg, W1, b1, g, bt, W2, b2):
    n, d = x.shape
    w1a = W1[:, :d].T
    w1b = W1[:, d:].T
    return pl.pallas_call(
        _mlp_body,
        out_shape=jax.ShapeDtypeStruct((n, d), jnp.float32),
    )(x, agg, w1a, w1b, b1.reshape(1, -1), g.reshape(1, -1),
      bt.reshape(1, -1), W2.T, b2.reshape(1, -1))


def kernel(var_feats, constr_feats, edge_index, edge_attr,
           W1, b1, g1, bt1, W2, b2, W3, b3, g2, bt2, W4, b4):
    n_edges = edge_index.shape[1]
    info = plsc.get_sparse_core_info()
    ns = info.num_subcores
    per_tile = -(-n_edges // (ns * 2 * CHUNK)) * 2 * CHUNK
    e_pad = per_tile * ns
    pad = e_pad - n_edges
    src = jnp.pad(edge_index[0], (0, pad))
    dst = jnp.pad(edge_index[1], (0, pad))
    attr = jnp.pad(edge_attr, (0, pad))

    constr_agg, var_agg = _message_pass(var_feats, constr_feats, src, dst, attr)
    var_updated = _mlp_update(var_feats, var_agg, W1, b1, g1, bt1, W2, b2)
    constr_updated = _mlp_update(constr_feats, constr_agg, W3, b3, g2, bt2, W4, b4)
    return (var_updated, constr_updated)
